# baseline (speedup 1.0000x reference)
import functools

import jax
import jax.numpy as jnp
from jax import lax
from jax.experimental import pallas as pl
from jax.experimental.pallas import tpu as pltpu
from jax.experimental.pallas import tpu_sc as plsc

NUM_CORES = 2
NUM_SUBCORES = 16
LANES = 16
NUM_WORKERS = NUM_CORES * NUM_SUBCORES
BATCH = 16384
BPW = BATCH // NUM_WORKERS
DIM = 16


def _sc_kernel(uidx_hbm, iidx_hbm, ut_hbm, it_hbm, w_hbm, out_hbm,
               uidx_v, iidx_v, urows_v, irows_v, out_v, w_v, sem):
    wid = lax.axis_index("s") * NUM_CORES + lax.axis_index("c")
    base = pl.multiple_of(wid * BPW, BPW)

    pltpu.sync_copy(uidx_hbm.at[pl.ds(base, BPW)], uidx_v)
    pltpu.sync_copy(iidx_hbm.at[pl.ds(base, BPW)], iidx_v)
    pltpu.sync_copy(w_hbm, w_v)

    cu = pltpu.async_copy(ut_hbm.at[uidx_v], urows_v, sem)
    ci = pltpu.async_copy(it_hbm.at[iidx_v], irows_v, sem)
    cu.wait()
    ci.wait()

    wvec = w_v[...]
    lane = lax.iota(jnp.int32, LANES)
    _dnums = lax.GatherDimensionNumbers(
        offset_dims=(), collapsed_slice_dims=(0,), start_index_map=(0,))

    def _shuffle(x, idx):
        return lax.gather(
            x, idx.reshape(LANES, 1), _dnums, slice_sizes=(1,),
            mode=lax.GatherScatterMode.PROMISE_IN_BOUNDS)

    rounds = [(lane ^ k, (lane & k) == 0) for k in (1, 2, 4, 8)]

    def tile(t, carry):
        vs = []
        for j in range(LANES):
            r = t * LANES + j
            vs.append(urows_v[r, :] * irows_v[r, :] * wvec)
        for sidx, mk in rounds:
            vs = [
                jnp.where(mk, a + _shuffle(a, sidx), b + _shuffle(b, sidx))
                for a, b in zip(vs[0::2], vs[1::2])
            ]
        s = vs[0]
        out_v[pl.ds(pl.multiple_of(t * LANES, LANES), LANES)] = (
            1.0 / (1.0 + jnp.exp(-s)))
        return carry

    lax.fori_loop(0, BPW // LANES, tile, 0)

    pltpu.sync_copy(out_v, out_hbm.at[pl.ds(base, BPW)])


@functools.partial(jax.jit, static_argnames=())
def _run(user_indices, item_indices, emb_user, emb_item, w_flat):
    mesh = plsc.VectorSubcoreMesh(core_axis_name="c", subcore_axis_name="s")
    return pl.kernel(
        _sc_kernel,
        out_type=jax.ShapeDtypeStruct((BATCH,), jnp.float32),
        mesh=mesh,
        compiler_params=pltpu.CompilerParams(use_tc_tiling_on_sc=False),
        scratch_types=[
            pltpu.VMEM((BPW,), jnp.int32),
            pltpu.VMEM((BPW,), jnp.int32),
            pltpu.VMEM((BPW, DIM), jnp.float32),
            pltpu.VMEM((BPW, DIM), jnp.float32),
            pltpu.VMEM((BPW,), jnp.float32),
            pltpu.VMEM((LANES,), jnp.float32),
            pltpu.SemaphoreType.DMA,
        ],
    )(user_indices, item_indices, emb_user, emb_item, w_flat)


def kernel(user_indices, item_indices, emb_user, emb_item, w_gmf):
    return _run(
        user_indices.astype(jnp.int32),
        item_indices.astype(jnp.int32),
        emb_user,
        emb_item,
        w_gmf.reshape(DIM),
    )


# speedup vs baseline: 6.1644x; 6.1644x over previous
import jax
import jax.numpy as jnp
from jax import lax
from jax.experimental import pallas as pl
from jax.experimental.pallas import tpu as pltpu
from jax.experimental.pallas import tpu_sc as plsc

NUM_CORES = 2
NUM_SUBCORES = 16
LANES = 16
NUM_WORKERS = NUM_CORES * NUM_SUBCORES
BATCH = 16384
BPW = BATCH // NUM_WORKERS
DIM = 16
NROWS = 1000000
CHUNK = 16
NCHUNKS = BPW // CHUNK


def _sc_kernel(uidx_hbm, iidx_hbm, ut_hbm, it_hbm, w_hbm, out_hbm,
               uidx_v, iidx_v, ublk_v, iblk_v, out_v, w_v, sem):
    wid = lax.axis_index("s") * NUM_CORES + lax.axis_index("c")
    base = pl.multiple_of(wid * BPW, BPW)

    pltpu.sync_copy(uidx_hbm.at[pl.ds(base, BPW)], uidx_v)
    pltpu.sync_copy(iidx_hbm.at[pl.ds(base, BPW)], iidx_v)
    pltpu.sync_copy(w_hbm, w_v)

    uv3 = ut_hbm.reshape(2, 8, NROWS)
    iv3 = it_hbm.reshape(2, 8, NROWS)

    wvec = w_v[...]
    lanes = lax.iota(jnp.int32, LANES)
    _dnums = lax.GatherDimensionNumbers(
        offset_dims=(), collapsed_slice_dims=(0,), start_index_map=(0,))

    def _bcast(x, d):
        return lax.gather(
            x, jnp.full((LANES, 1), d, jnp.int32), _dnums, slice_sizes=(1,),
            mode=lax.GatherScatterMode.PROMISE_IN_BOUNDS)

    wb = [_bcast(wvec, d) for d in range(DIM)]

    ublk2 = ublk_v.reshape(CHUNK * DIM, 128)
    iblk2 = iblk_v.reshape(CHUNK * DIM, 128)

    def chunk_body(c, carry):
        off = pl.multiple_of(c * CHUNK, CHUNK)
        uvec = uidx_v[pl.ds(off, CHUNK)]
        ivec = iidx_v[pl.ds(off, CHUNK)]
        ustart = (uvec >> 7) << 7
        istart = (ivec >> 7) << 7
        copies = []
        for j in range(CHUNK):
            uo = pl.multiple_of(ustart[j], 128)
            io = pl.multiple_of(istart[j], 128)
            copies.append(pltpu.async_copy(
                uv3.at[:, :, pl.ds(uo, 128)], ublk_v.at[j], sem))
            copies.append(pltpu.async_copy(
                iv3.at[:, :, pl.ds(io, 128)], iblk_v.at[j], sem))
        for cp in copies:
            cp.wait()

        um = uvec & 127
        im = ivec & 127
        row_base = lanes * DIM
        acc = None
        for d in range(DIM):
            ug = plsc.load_gather(ublk2, [row_base + d, um])
            ig = plsc.load_gather(iblk2, [row_base + d, im])
            term = ug * ig * wb[d]
            acc = term if acc is None else acc + term
        out_v[pl.ds(off, CHUNK)] = 1.0 / (1.0 + jnp.exp(-acc))
        return carry

    lax.fori_loop(0, NCHUNKS, chunk_body, 0)

    pltpu.sync_copy(out_v, out_hbm.at[pl.ds(base, BPW)])


@jax.jit
def _run(user_indices, item_indices, emb_user_t, emb_item_t, w_flat):
    mesh = plsc.VectorSubcoreMesh(core_axis_name="c", subcore_axis_name="s")
    return pl.kernel(
        _sc_kernel,
        out_type=jax.ShapeDtypeStruct((BATCH,), jnp.float32),
        mesh=mesh,
        compiler_params=pltpu.CompilerParams(needs_layout_passes=False),
        scratch_types=[
            pltpu.VMEM((BPW,), jnp.int32),
            pltpu.VMEM((BPW,), jnp.int32),
            pltpu.VMEM((CHUNK, 2, 8, 128), jnp.float32),
            pltpu.VMEM((CHUNK, 2, 8, 128), jnp.float32),
            pltpu.VMEM((BPW,), jnp.float32),
            pltpu.VMEM((LANES,), jnp.float32),
            pltpu.SemaphoreType.DMA,
        ],
    )(user_indices, item_indices, emb_user_t, emb_item_t, w_flat)


def kernel(user_indices, item_indices, emb_user, emb_item, w_gmf):
    return _run(
        user_indices.astype(jnp.int32),
        item_indices.astype(jnp.int32),
        emb_user.T,
        emb_item.T,
        w_gmf.reshape(DIM),
    )
